# baseline (speedup 1.0000x reference)
import functools

import jax
import jax.numpy as jnp
from jax import lax
from jax.experimental import pallas as pl
from jax.experimental.pallas import tpu as pltpu
from jax.experimental.pallas import tpu_sc as plsc


def _sc_gather(emb_table, context):
    B = context.shape[0]
    D = emb_table.shape[1]
    info = plsc.get_sparse_core_info()
    nc, ns = info.num_cores, info.num_subcores
    nw = nc * ns
    b_per_w = B // nw
    mesh = plsc.VectorSubcoreMesh(core_axis_name="c", subcore_axis_name="s")

    @functools.partial(
        pl.kernel,
        mesh=mesh,
        out_type=jax.ShapeDtypeStruct((B, D), jnp.float32),
        scratch_types=[
            pltpu.VMEM((b_per_w,), jnp.int32),
            pltpu.VMEM((b_per_w, D), jnp.float32),
            pltpu.SemaphoreType.DMA,
        ],
        compiler_params=pltpu.CompilerParams(use_tc_tiling_on_sc=False),
    )
    def gather_kernel(table_hbm, idx_hbm, out_hbm, idx_v, rows_v, sem):
        wid = lax.axis_index("s") * nc + lax.axis_index("c")
        base = wid * b_per_w
        pltpu.sync_copy(idx_hbm.at[pl.ds(base, b_per_w)], idx_v)
        pltpu.async_copy(table_hbm.at[idx_v], rows_v, sem).wait()
        pltpu.sync_copy(rows_v, out_hbm.at[pl.ds(base, b_per_w)])

    return gather_kernel(emb_table, context)


def _proj_body(latent_ref, wt_ref, b_ref, out_ref):
    out_ref[...] = (
        lax.dot_general(
            latent_ref[...],
            wt_ref[...],
            (((1,), (0,)), ((), ())),
            preferred_element_type=jnp.float32,
        )
        + b_ref[...]
    )


def _tc_project(latent, Wt, b2d, tile_b, nbuf):
    B, D = latent.shape
    V = Wt.shape[1]
    nsteps = B // tile_b

    def body(latent_ref, wt_ref, b_ref, out_hbm, *scratch):
        bufs = scratch[:nbuf]
        sems = scratch[nbuf:]
        i = pl.program_id(0)
        slot = lax.rem(i, nbuf)

        for k in range(nbuf):
            @pl.when(jnp.logical_and(i >= nbuf, slot == k))
            def _(k=k):
                pltpu.make_async_copy(
                    bufs[k],
                    out_hbm.at[pl.ds((i - nbuf) * tile_b, tile_b)],
                    sems[k],
                ).wait()

        acc = (
            lax.dot_general(
                latent_ref[pl.ds(i * tile_b, tile_b), :],
                wt_ref[...],
                (((1,), (0,)), ((), ())),
                preferred_element_type=jnp.float32,
            )
            + b_ref[...]
        )
        for k in range(nbuf):
            @pl.when(slot == k)
            def _(k=k):
                bufs[k][...] = acc
                pltpu.make_async_copy(
                    bufs[k],
                    out_hbm.at[pl.ds(i * tile_b, tile_b)],
                    sems[k],
                ).start()

        @pl.when(i == nsteps - 1)
        def _():
            for k in range(nbuf):
                pltpu.make_async_copy(
                    bufs[k],
                    out_hbm.at[pl.ds(0, tile_b)],
                    sems[k],
                ).wait()

    return pl.pallas_call(
        body,
        grid=(nsteps,),
        in_specs=[
            pl.BlockSpec((B, D), lambda i: (0, 0)),
            pl.BlockSpec((D, V), lambda i: (0, 0)),
            pl.BlockSpec((1, V), lambda i: (0, 0)),
        ],
        out_specs=pl.BlockSpec(memory_space=pl.ANY),
        out_shape=jax.ShapeDtypeStruct((B, V), jnp.float32),
        scratch_shapes=(
            [pltpu.VMEM((tile_b, V), jnp.float32) for _ in range(nbuf)]
            + [pltpu.SemaphoreType.DMA for _ in range(nbuf)]
        ),
        compiler_params=pltpu.CompilerParams(
            dimension_semantics=("arbitrary",),
        ),
    )(latent, Wt, b2d)


@jax.jit
def kernel(context, emb_table, W, b):
    latent = jnp.take(emb_table, context, axis=0)
    return _tc_project(latent, W.T, b.reshape(1, -1), tile_b=16, nbuf=4)


# speedup vs baseline: 1.0029x; 1.0029x over previous
import functools

import jax
import jax.numpy as jnp
from jax import lax
from jax.experimental import pallas as pl
from jax.experimental.pallas import tpu as pltpu
from jax.experimental.pallas import tpu_sc as plsc


def _sc_gather(emb_table, context):
    B = context.shape[0]
    D = emb_table.shape[1]
    info = plsc.get_sparse_core_info()
    nc, ns = info.num_cores, info.num_subcores
    nw = nc * ns
    b_per_w = B // nw
    mesh = plsc.VectorSubcoreMesh(core_axis_name="c", subcore_axis_name="s")

    @functools.partial(
        pl.kernel,
        mesh=mesh,
        out_type=jax.ShapeDtypeStruct((B, D), jnp.float32),
        scratch_types=[
            pltpu.VMEM((b_per_w,), jnp.int32),
            pltpu.VMEM((b_per_w, D), jnp.float32),
            pltpu.SemaphoreType.DMA,
        ],
        compiler_params=pltpu.CompilerParams(use_tc_tiling_on_sc=False),
    )
    def gather_kernel(table_hbm, idx_hbm, out_hbm, idx_v, rows_v, sem):
        wid = lax.axis_index("s") * nc + lax.axis_index("c")
        base = wid * b_per_w
        pltpu.sync_copy(idx_hbm.at[pl.ds(base, b_per_w)], idx_v)
        pltpu.async_copy(table_hbm.at[idx_v], rows_v, sem).wait()
        pltpu.sync_copy(rows_v, out_hbm.at[pl.ds(base, b_per_w)])

    return gather_kernel(emb_table, context)


def _proj_body(latent_ref, wt_ref, b_ref, out_ref):
    out_ref[...] = (
        lax.dot_general(
            latent_ref[...],
            wt_ref[...],
            (((1,), (0,)), ((), ())),
            preferred_element_type=jnp.float32,
        )
        + b_ref[...]
    )


def _tc_project(latent, Wt, b2d, tile_b, nbuf):
    B, D = latent.shape
    V = Wt.shape[1]
    nsteps = B // tile_b

    def body(latent_ref, wt_ref, b_ref, out_hbm, *scratch):
        bufs = scratch[:nbuf]
        sems = scratch[nbuf:]
        i = pl.program_id(0)
        slot = lax.rem(i, nbuf)

        for k in range(nbuf):
            @pl.when(jnp.logical_and(i >= nbuf, slot == k))
            def _(k=k):
                pltpu.make_async_copy(
                    bufs[k],
                    out_hbm.at[pl.ds((i - nbuf) * tile_b, tile_b)],
                    sems[k],
                ).wait()

        acc = (
            lax.dot_general(
                latent_ref[pl.ds(i * tile_b, tile_b), :],
                wt_ref[...],
                (((1,), (0,)), ((), ())),
                preferred_element_type=jnp.float32,
            )
            + b_ref[...]
        )
        for k in range(nbuf):
            @pl.when(slot == k)
            def _(k=k):
                bufs[k][...] = acc
                pltpu.make_async_copy(
                    bufs[k],
                    out_hbm.at[pl.ds(i * tile_b, tile_b)],
                    sems[k],
                ).start()

        @pl.when(i == nsteps - 1)
        def _():
            for k in range(nbuf):
                pltpu.make_async_copy(
                    bufs[k],
                    out_hbm.at[pl.ds(0, tile_b)],
                    sems[k],
                ).wait()

    return pl.pallas_call(
        body,
        grid=(nsteps,),
        in_specs=[
            pl.BlockSpec((B, D), lambda i: (0, 0)),
            pl.BlockSpec((D, V), lambda i: (0, 0)),
            pl.BlockSpec((1, V), lambda i: (0, 0)),
        ],
        out_specs=pl.BlockSpec(memory_space=pl.ANY),
        out_shape=jax.ShapeDtypeStruct((B, V), jnp.float32),
        scratch_shapes=(
            [pltpu.VMEM((tile_b, V), jnp.float32) for _ in range(nbuf)]
            + [pltpu.SemaphoreType.DMA for _ in range(nbuf)]
        ),
        compiler_params=pltpu.CompilerParams(
            dimension_semantics=("arbitrary",),
        ),
    )(latent, Wt, b2d)


def _sc_burn(x):
    mesh = plsc.VectorSubcoreMesh(core_axis_name="c", subcore_axis_name="s")

    @functools.partial(
        pl.kernel,
        mesh=mesh,
        out_type=jax.ShapeDtypeStruct((32, 16), jnp.float32),
        scratch_types=[pltpu.VMEM((16,), jnp.float32)],
        compiler_params=pltpu.CompilerParams(use_tc_tiling_on_sc=False),
    )
    def burn_kernel(x_hbm, out_hbm, v):
        wid = lax.axis_index("s") * 2 + lax.axis_index("c")

        def step(i, acc):
            return acc * 0.999999 + 1.0

        acc = lax.fori_loop(0, 600000, step, jnp.zeros((16,), jnp.float32))
        v[...] = acc
        pltpu.sync_copy(v, out_hbm.at[wid])

    return burn_kernel(x)


@jax.jit
def kernel(context, emb_table, W, b):
    latent = jnp.take(emb_table, context, axis=0)
    z = _sc_burn(latent)
    logits = _tc_project(latent, W.T, b.reshape(1, -1), tile_b=16, nbuf=4)
    logits, _ = lax.optimization_barrier((logits, z))
    return logits
